# baseline (speedup 1.0000x reference)
import jax
import jax.numpy as jnp
from jax import lax
from jax.experimental import pallas as pl
from jax.experimental.pallas import tpu as pltpu

N_EXP = 16
BLK = 1024


def _router_body(x_ref, w_ref, b_ref, wout_ref, iout_ref, cv_ref,
                 lt2_ref, esum_ref):
    i = pl.program_id(0)
    nb = pl.num_programs(0) - 1
    par = lax.rem(i, 2)

    @pl.when(i == 0)
    def _init():
        esum_ref[...] = jnp.zeros_like(esum_ref)
        lt2_ref[...] = jnp.zeros_like(lt2_ref)

    logits = jnp.dot(x_ref[...], w_ref[...], preferred_element_type=jnp.float32)
    lt2_ref[par, :, :] = logits.T + b_ref[...]

    lt = lt2_ref[1 - par, :, :]
    m = jnp.max(lt, axis=0, keepdims=True)
    e = jnp.exp(lt - m)
    s = jnp.sum(e, axis=0, keepdims=True)
    p = e / s

    psum = jnp.sum(p, axis=1, keepdims=True)
    esum_ref[...] += jnp.where(i > 0, psum, 0.0)

    iota = lax.broadcasted_iota(jnp.int32, (N_EXP, BLK), 0)
    m1 = jnp.max(p, axis=0, keepdims=True)
    i1 = jnp.min(jnp.where(p == m1, iota, N_EXP), axis=0, keepdims=True)
    p2 = jnp.where(iota == i1, -1.0, p)
    m2 = jnp.max(p2, axis=0, keepdims=True)
    i2 = jnp.min(jnp.where(p2 == m2, iota, N_EXP), axis=0, keepdims=True)

    tot = m1 + m2
    pack = jnp.concatenate(
        [m1 / tot, m2 / tot, i1.astype(jnp.float32), i2.astype(jnp.float32),
         tot, tot, tot, tot], axis=0)
    packt = pack.T
    wout_ref[...] = packt[:, 0:2]
    iout_ref[...] = packt[:, 2:4].astype(jnp.int32)

    @pl.when(i == nb)
    def _finish():
        sums = esum_ref[...]
        f = sums / jnp.sum(sums)
        mean = jnp.sum(f) / N_EXP
        var = jnp.sum((f - mean) ** 2) / N_EXP
        cv_ref[...] = jnp.sqrt(var).reshape(1, 1) / mean


def kernel(x, W, b):
    B, T, d = x.shape
    n = B * T
    x_flat = x.reshape(n, d)
    b2 = b.reshape(N_EXP, 1)
    nblk = n // BLK

    wout, iout, cv = pl.pallas_call(
        _router_body,
        grid=(nblk + 1,),
        in_specs=[
            pl.BlockSpec((BLK, d), lambda i: (jnp.minimum(i, nblk - 1), 0)),
            pl.BlockSpec((d, N_EXP), lambda i: (0, 0)),
            pl.BlockSpec((N_EXP, 1), lambda i: (0, 0)),
        ],
        out_specs=[
            pl.BlockSpec((BLK, 2), lambda i: (jnp.maximum(i - 1, 0), 0)),
            pl.BlockSpec((BLK, 2), lambda i: (jnp.maximum(i - 1, 0), 0)),
            pl.BlockSpec((1, 1), lambda i: (0, 0)),
        ],
        out_shape=[
            jax.ShapeDtypeStruct((n, 2), jnp.float32),
            jax.ShapeDtypeStruct((n, 2), jnp.int32),
            jax.ShapeDtypeStruct((1, 1), jnp.float32),
        ],
        scratch_shapes=[
            pltpu.VMEM((2, N_EXP, BLK), jnp.float32),
            pltpu.VMEM((N_EXP, 1), jnp.float32),
        ],
    )(x_flat, W, b2)

    return (wout.reshape(B, T, 2), iout.reshape(B, T, 2), cv.reshape(()))


# speedup vs baseline: 1.3825x; 1.3825x over previous
import jax
import jax.numpy as jnp
from jax import lax
from jax.experimental import pallas as pl
from jax.experimental.pallas import tpu as pltpu

N_EXP = 16
BLK = 1024
LCH = 128


def _router_body(x_ref, w_ref, b_ref, out_ref, cv_ref, esum_ref):
    i = pl.program_id(0)
    nblk = pl.num_programs(0)

    @pl.when(i == 0)
    def _init():
        esum_ref[...] = jnp.zeros_like(esum_ref)

    logits = jnp.dot(x_ref[...], w_ref[...], preferred_element_type=jnp.float32)
    lt = logits.T + b_ref[...]

    m = jnp.max(lt, axis=0, keepdims=True)
    e = jnp.exp(lt - m)
    s = jnp.sum(e, axis=0, keepdims=True)
    p = e / s

    pc = p.reshape(N_EXP, BLK // LCH, LCH)
    esum_ref[...] += jnp.sum(pc, axis=1)

    iota = lax.broadcasted_iota(jnp.int32, (N_EXP, BLK), 0)
    eq1 = lt == m
    i1 = jnp.min(jnp.where(eq1, iota, N_EXP), axis=0, keepdims=True)
    l2 = jnp.max(jnp.where(eq1, -jnp.inf, lt), axis=0, keepdims=True)
    i2 = jnp.min(jnp.where(lt == l2, iota, N_EXP), axis=0, keepdims=True)

    e2 = jnp.exp(l2 - m)
    w1 = 1.0 / (1.0 + e2)
    w2 = e2 * w1

    out_ref[0:1, :] = w1
    out_ref[1:2, :] = w2
    out_ref[2:3, :] = i1.astype(jnp.float32)
    out_ref[3:4, :] = i2.astype(jnp.float32)

    @pl.when(i == nblk - 1)
    def _finish():
        sums = jnp.sum(esum_ref[...], axis=1, keepdims=True)
        f = sums / jnp.sum(sums)
        mean = jnp.sum(f) / N_EXP
        var = jnp.sum((f - mean) ** 2) / N_EXP
        cv_ref[...] = jnp.sqrt(var).reshape(1, 1) / mean


def kernel(x, W, b):
    B, T, d = x.shape
    n = B * T
    x_flat = x.reshape(n, d)
    b2 = b.reshape(N_EXP, 1)
    nblk = n // BLK

    out_t, cv = pl.pallas_call(
        _router_body,
        grid=(nblk,),
        in_specs=[
            pl.BlockSpec((BLK, d), lambda i: (i, 0)),
            pl.BlockSpec((d, N_EXP), lambda i: (0, 0)),
            pl.BlockSpec((N_EXP, 1), lambda i: (0, 0)),
        ],
        out_specs=[
            pl.BlockSpec((4, BLK), lambda i: (0, i)),
            pl.BlockSpec((1, 1), lambda i: (0, 0)),
        ],
        out_shape=[
            jax.ShapeDtypeStruct((4, n), jnp.float32),
            jax.ShapeDtypeStruct((1, 1), jnp.float32),
        ],
        scratch_shapes=[pltpu.VMEM((N_EXP, LCH), jnp.float32)],
    )(x_flat, W, b2)

    o = out_t.T
    wout = o[:, 0:2].reshape(B, T, 2)
    iout = o[:, 2:4].astype(jnp.int32).reshape(B, T, 2)
    return (wout, iout, cv.reshape(()))
